# baseline (speedup 1.0000x reference)
import functools

import jax
import jax.numpy as jnp
from jax import lax
from jax.experimental import pallas as pl
from jax.experimental.pallas import tpu as pltpu
from jax.experimental.pallas import tpu_sc as plsc

D = 32
B = 4096
NC = 2
NS = 16
L = 16
NW = NC * NS
BPW = B // NW

_mesh = plsc.VectorSubcoreMesh(core_axis_name="c", subcore_axis_name="s")


@functools.partial(
    pl.kernel,
    mesh=_mesh,
    out_type=jax.ShapeDtypeStruct((B,), jnp.float32),
    scratch_types=[
        pltpu.VMEM((BPW,), jnp.int32),
        pltpu.VMEM((BPW,), jnp.int32),
        pltpu.VMEM((BPW, D), jnp.float32),
        pltpu.VMEM((BPW, D), jnp.float32),
        pltpu.VMEM((BPW,), jnp.float32),
        pltpu.SemaphoreType.DMA,
    ],
    compiler_params=pltpu.CompilerParams(
        needs_layout_passes=False, use_tc_tiling_on_sc=False),
)
def _negdot(idxw_hbm, idxc_hbm, word_hbm, ctx_hbm, out_hbm,
            idxw_v, idxc_v, wrows_v, crows_v, out_v, sem):
    wid = lax.axis_index("s") * NC + lax.axis_index("c")
    base = wid * BPW

    pltpu.sync_copy(idxw_hbm.at[pl.ds(base, BPW)], idxw_v)
    pltpu.sync_copy(idxc_hbm.at[pl.ds(base, BPW)], idxc_v)

    cp_w = pltpu.async_copy(word_hbm.at[idxw_v], wrows_v, sem)
    cp_c = pltpu.async_copy(ctx_hbm.at[idxc_v], crows_v, sem)
    cp_w.wait()
    cp_c.wait()

    for g in range(BPW // L):
        rows = jnp.full((L,), g * L, jnp.int32) + lax.iota(jnp.int32, L)
        acc = jnp.zeros((L,), jnp.float32)
        for d in range(D):
            col = jnp.full((L,), d, jnp.int32)
            w = plsc.load_gather(wrows_v, [rows, col])
            c = plsc.load_gather(crows_v, [rows, col])
            acc = acc + w * c
        out_v[pl.ds(g * L, L)] = acc

    pltpu.sync_copy(out_v, out_hbm.at[pl.ds(base, BPW)])


def kernel(inputs, word_embeddings, context_embeddings):
    idx_word = inputs[:, 1].astype(jnp.int32)
    idx_ctx = inputs[:, 0].astype(jnp.int32)
    return _negdot(idx_word, idx_ctx, word_embeddings, context_embeddings)
